# baseline (speedup 1.0000x reference)
import math

import jax
import jax.numpy as jnp
import numpy as np
from jax import lax
from jax.experimental import pallas as pl
from jax.experimental.pallas import tpu as pltpu

SEQ = 16
DIM = 32
NQ = 3
NWORD = 32
WIN = 2
KTAP = 2 * WIN + 1
LCNN = 3
LOUT = 3
LN_EPS = 1e-5


def _layernorm(x, g, b):
    mu = jnp.mean(x, axis=-1, keepdims=True)
    var = jnp.mean((x - mu) ** 2, axis=-1, keepdims=True)
    return (x - mu) * lax.rsqrt(var + LN_EPS) * g + b


def _gelu(x):
    return 0.5 * x * (1.0 + lax.erf(x * 0.7071067811865476))


def _band_cat(conv_w):
    w = conv_w.reshape(LCNN, KTAP, KTAP)
    c = jnp.arange(DIM)[:, None]
    d = jnp.arange(DIM)[None, :]
    dj = c - d + WIN
    valid = (dj >= 0) & (dj < KTAP)
    djc = jnp.clip(dj, 0, KTAP - 1)
    band = jnp.where(valid[None, None], w[:, :, djc], 0.0)
    return band.transpose(0, 2, 1, 3).reshape(LCNN, DIM, KTAP * DIM)


def _shift_sum_t(g, bb):
    n = SEQ * bb
    acc = None
    for di in range(KTAP):
        sl = g[di * DIM:(di + 1) * DIM, :]
        sh = (di - WIN) * bb
        if sh > 0:
            z = jnp.zeros((DIM, sh), jnp.float32)
            t = jnp.concatenate([sl[:, sh:], z], axis=1)
        elif sh < 0:
            z = jnp.zeros((DIM, -sh), jnp.float32)
            t = jnp.concatenate([z, sl[:, :n + sh]], axis=1)
        else:
            t = sl
        acc = t if acc is None else acc + t
    return acc


def _dti_block_kernel(phar_ref, mol_ref, prot_ref, packt_ref, matsn_ref,
                      vec_ref, vect_ref, packh_ref, out_ref):
    bb = phar_ref.shape[0]
    n = SEQ * bb
    f32 = jnp.float32

    idx = prot_ref[0]
    iota = lax.broadcasted_iota(jnp.int32, (NWORD, n), 0)
    onehot = (idx == iota).astype(jnp.bfloat16)
    t1t = packt_ref[0:KTAP * DIM, :]
    g = jnp.dot(t1t, onehot, preferred_element_type=f32)
    ba_col = vect_ref[:, 0:1]
    xs = jnp.maximum(_shift_sum_t(g, bb) + vect_ref[:, 1:2], 0.0)
    xs = xs.astype(jnp.bfloat16)
    for l in range(1, LCNN):
        bt = packt_ref[l * KTAP * DIM:(l + 1) * KTAP * DIM, :]
        g = jnp.dot(bt, xs, preferred_element_type=f32)
        xs = jnp.maximum(_shift_sum_t(g, bb) + vect_ref[:, 1 + l:2 + l], 0.0)
        xs = xs.astype(jnp.bfloat16)

    p = phar_ref[...].astype(jnp.bfloat16)
    h1 = _gelu(jnp.dot(p, matsn_ref[0:NQ * DIM, :],
                       preferred_element_type=f32) + vec_ref[0:1, :])
    h1 = h1.astype(jnp.bfloat16)
    prompt = jnp.dot(h1, matsn_ref[NQ * DIM:NQ * DIM + DIM, :],
                     preferred_element_type=f32) + vec_ref[1:2, :]
    mol = _layernorm(prompt + mol_ref[...], vec_ref[2:3, :], vec_ref[3:4, :])
    molt = jnp.transpose(mol).astype(jnp.bfloat16)

    wat = packt_ref[LCNN * KTAP * DIM:LCNN * KTAP * DIM + DIM, :]
    ht = jnp.maximum(jnp.dot(wat, molt, preferred_element_type=f32)
                     + ba_col, 0.0)
    hst = jnp.maximum(jnp.dot(wat, xs, preferred_element_type=f32)
                      + ba_col, 0.0)
    ht_tiled = jnp.concatenate([ht] * SEQ, axis=1)
    ones_d = jnp.ones((1, DIM), f32)
    sig = jnp.dot(ones_d, ht_tiled * hst, preferred_element_type=f32)
    wts = jnp.tanh(sig)
    wprod = wts * hst
    prott = wprod[:, 0:bb]
    for s in range(1, SEQ):
        prott = prott + wprod[:, s * bb:(s + 1) * bb]
    prott = (prott * (1.0 / SEQ)).astype(jnp.bfloat16)

    D2 = 2 * DIM
    cat = jnp.maximum(
        jnp.dot(packh_ref[0:D2, 0:DIM], molt, preferred_element_type=f32)
        + jnp.dot(packh_ref[0:D2, DIM:D2], prott, preferred_element_type=f32)
        + packh_ref[LOUT * D2:LOUT * D2 + D2, 0:1].astype(f32), 0.0)
    cat = cat.astype(jnp.bfloat16)
    for j in range(1, LOUT):
        wjt = packh_ref[j * D2:(j + 1) * D2, :]
        cat = jnp.maximum(
            jnp.dot(wjt, cat, preferred_element_type=f32)
            + packh_ref[LOUT * D2:LOUT * D2 + D2, j:j + 1].astype(f32),
            0.0).astype(jnp.bfloat16)

    ones_2d = jnp.ones((1, D2), jnp.bfloat16)
    wint_col = packh_ref[LOUT * D2:LOUT * D2 + D2, LOUT:LOUT + 1]
    out = (jnp.dot(ones_2d, cat * wint_col, preferred_element_type=f32)
           + vec_ref[4:5, 0:1])
    out_ref[...] = out


@jax.jit
def _forward(phar_prompt, mol_repr, protein_batch, proj_w1, proj_b1, proj_w2,
             proj_b2, emb, mol_gamma, mol_beta, prot_gamma, prot_beta, conv_w,
             conv_b, wa, ba, wout_w, wout_b, wint_w, wint_b):
    bn = mol_repr.shape[0]
    bb = math.gcd(bn, 512)
    nblk = bn // bb

    phar2 = phar_prompt.reshape(bn, NQ * DIM)
    prot_flat = protein_batch.reshape(nblk, bb, SEQ).transpose(0, 2, 1) \
                             .reshape(nblk, 1, SEQ * bb)

    band = _band_cat(conv_w)
    emb_ln = _layernorm(emb, prot_gamma, prot_beta)
    t1 = jnp.dot(emb_ln, band[0])
    packt = jnp.concatenate([
        t1.T,
        band[1].T, band[2].T,
        wa.T,
    ], axis=0).astype(jnp.bfloat16)

    matsn = jnp.concatenate([proj_w1, proj_w2],
                            axis=0).astype(jnp.bfloat16)
    vec = jnp.concatenate([
        proj_b1, proj_b2, mol_gamma, mol_beta,
        jnp.pad(wint_b, ((0, 0), (0, DIM - 1))),
    ], axis=0)
    vect = jnp.concatenate([
        ba.T,
        jnp.broadcast_to(conv_b[0], (DIM, 1)),
        jnp.broadcast_to(conv_b[1], (DIM, 1)),
        jnp.broadcast_to(conv_b[2], (DIM, 1)),
        jnp.zeros((DIM, 1), jnp.float32),
    ], axis=1)

    D2 = 2 * DIM
    headmats = jnp.concatenate(
        [wout_w[j].T for j in range(LOUT)], axis=0)
    bias_cols = jnp.concatenate(
        [wout_b[j].T for j in range(LOUT)] + [wint_w], axis=1)
    packh = jnp.concatenate([
        headmats,
        jnp.pad(bias_cols, ((0, 0), (0, D2 - (LOUT + 1)))),
    ], axis=0).astype(jnp.bfloat16)

    out = pl.pallas_call(
        _dti_block_kernel,
        out_shape=jax.ShapeDtypeStruct((1, bn), jnp.float32),
        grid=(nblk,),
        in_specs=[
            pl.BlockSpec((bb, NQ * DIM), lambda b: (b, 0)),
            pl.BlockSpec((bb, DIM), lambda b: (b, 0)),
            pl.BlockSpec((1, 1, SEQ * bb), lambda b: (b, 0, 0)),
            pl.BlockSpec((LCNN * KTAP * DIM + DIM, DIM), lambda b: (0, 0)),
            pl.BlockSpec(((NQ + 1) * DIM, DIM), lambda b: (0, 0)),
            pl.BlockSpec((5, DIM), lambda b: (0, 0)),
            pl.BlockSpec((DIM, 5), lambda b: (0, 0)),
            pl.BlockSpec((4 * D2, D2), lambda b: (0, 0)),
        ],
        out_specs=pl.BlockSpec((1, bb), lambda b: (0, b)),
        compiler_params=pltpu.CompilerParams(
            dimension_semantics=("parallel",)),
    )(phar2, mol_repr, prot_flat, packt, matsn, vec, vect, packh)
    return out.reshape(bn, 1)


def kernel(phar_prompt, mol_repr, protein_batch, proj_w1, proj_b1, proj_w2,
           proj_b2, emb, mol_gamma, mol_beta, prot_gamma, prot_beta, conv_w,
           conv_b, wa, ba, wout_w, wout_b, wint_w, wint_b):
    return _forward(phar_prompt, mol_repr, protein_batch, proj_w1, proj_b1,
                    proj_w2, proj_b2, emb, mol_gamma, mol_beta, prot_gamma,
                    prot_beta, conv_w, conv_b, wa, ba, wout_w, wout_b,
                    wint_w, wint_b)


# speedup vs baseline: 1.1318x; 1.1318x over previous
import math

import jax
import jax.numpy as jnp
import numpy as np
from jax import lax
from jax.experimental import pallas as pl
from jax.experimental.pallas import tpu as pltpu

SEQ = 16
DIM = 32
NQ = 3
NWORD = 32
WIN = 2
KTAP = 2 * WIN + 1
LCNN = 3
LOUT = 3
LN_EPS = 1e-5


def _layernorm(x, g, b):
    mu = jnp.mean(x, axis=-1, keepdims=True)
    var = jnp.mean((x - mu) ** 2, axis=-1, keepdims=True)
    return (x - mu) * lax.rsqrt(var + LN_EPS) * g + b


def _gelu(x):
    return 0.5 * x * (1.0 + lax.erf(x * 0.7071067811865476))


def _band_cat(conv_w):
    w = conv_w.reshape(LCNN, KTAP, KTAP)
    c = jnp.arange(DIM)[:, None]
    d = jnp.arange(DIM)[None, :]
    dj = c - d + WIN
    valid = (dj >= 0) & (dj < KTAP)
    djc = jnp.clip(dj, 0, KTAP - 1)
    band = jnp.where(valid[None, None], w[:, :, djc], 0.0)
    return band.transpose(0, 2, 1, 3).reshape(LCNN, DIM, KTAP * DIM)


def _shift_sum_t(g, bb):
    n = SEQ * bb
    acc = None
    for di in range(KTAP):
        sl = g[di * DIM:(di + 1) * DIM, :]
        sh = (di - WIN) * bb
        if sh > 0:
            z = jnp.zeros((DIM, sh), jnp.float32)
            t = jnp.concatenate([sl[:, sh:], z], axis=1)
        elif sh < 0:
            z = jnp.zeros((DIM, -sh), jnp.float32)
            t = jnp.concatenate([z, sl[:, :n + sh]], axis=1)
        else:
            t = sl
        acc = t if acc is None else acc + t
    return acc


def _dti_block_kernel(phar_ref, mol_ref, prot_ref, packt_ref, matsn_ref,
                      vec_ref, vect_ref, packh_ref, out_ref):
    bb = phar_ref.shape[0]
    n = SEQ * bb
    f32 = jnp.float32

    idx = prot_ref[0]
    iota = lax.broadcasted_iota(jnp.int32, (NWORD, n), 0)
    onehot = (idx == iota).astype(jnp.bfloat16)
    t1t = packt_ref[0:KTAP * DIM, :]
    g = jnp.dot(t1t, onehot, preferred_element_type=f32)
    ba_col = vect_ref[:, 0:1]
    xs = jnp.maximum(_shift_sum_t(g, bb) + vect_ref[:, 1:2], 0.0)
    xs = xs.astype(jnp.bfloat16)
    for l in range(1, LCNN):
        bt = packt_ref[l * KTAP * DIM:(l + 1) * KTAP * DIM, :]
        g = jnp.dot(bt, xs, preferred_element_type=f32)
        xs = jnp.maximum(_shift_sum_t(g, bb) + vect_ref[:, 1 + l:2 + l], 0.0)
        xs = xs.astype(jnp.bfloat16)

    p = phar_ref[...].astype(jnp.bfloat16)
    h1 = _gelu(jnp.dot(p, matsn_ref[0:NQ * DIM, :],
                       preferred_element_type=f32) + vec_ref[0:1, :])
    h1 = h1.astype(jnp.bfloat16)
    prompt = jnp.dot(h1, matsn_ref[NQ * DIM:NQ * DIM + DIM, :],
                     preferred_element_type=f32) + vec_ref[1:2, :]
    mol = _layernorm(prompt + mol_ref[...], vec_ref[2:3, :], vec_ref[3:4, :])
    molt = jnp.transpose(mol).astype(jnp.bfloat16)

    wat = packt_ref[LCNN * KTAP * DIM:LCNN * KTAP * DIM + DIM, :]
    ht = jnp.maximum(jnp.dot(wat, molt, preferred_element_type=f32)
                     + ba_col, 0.0)
    hst = jnp.maximum(jnp.dot(wat, xs, preferred_element_type=f32)
                      + ba_col, 0.0)
    ht_tiled = jnp.concatenate([ht] * SEQ, axis=1)
    ones_d = jnp.ones((1, DIM), f32)
    sig = jnp.dot(ones_d, ht_tiled * hst, preferred_element_type=f32)
    wts = jnp.tanh(sig)
    wprod = wts * hst
    prott = wprod[:, 0:bb]
    for s in range(1, SEQ):
        prott = prott + wprod[:, s * bb:(s + 1) * bb]
    prott = (prott * (1.0 / SEQ)).astype(jnp.bfloat16)

    D2 = 2 * DIM
    cat = jnp.maximum(
        jnp.dot(packh_ref[0:D2, 0:DIM], molt, preferred_element_type=f32)
        + jnp.dot(packh_ref[0:D2, DIM:D2], prott, preferred_element_type=f32)
        + packh_ref[LOUT * D2:LOUT * D2 + D2, 0:1].astype(f32), 0.0)
    cat = cat.astype(jnp.bfloat16)
    for j in range(1, LOUT):
        wjt = packh_ref[j * D2:(j + 1) * D2, :]
        cat = jnp.maximum(
            jnp.dot(wjt, cat, preferred_element_type=f32)
            + packh_ref[LOUT * D2:LOUT * D2 + D2, j:j + 1].astype(f32),
            0.0).astype(jnp.bfloat16)

    ones_2d = jnp.ones((1, D2), jnp.bfloat16)
    wint_col = packh_ref[LOUT * D2:LOUT * D2 + D2, LOUT:LOUT + 1]
    out = (jnp.dot(ones_2d, cat * wint_col, preferred_element_type=f32)
           + vec_ref[4:5, 0:1])
    out_ref[...] = out


@jax.jit
def _forward(phar_prompt, mol_repr, protein_batch, proj_w1, proj_b1, proj_w2,
             proj_b2, emb, mol_gamma, mol_beta, prot_gamma, prot_beta, conv_w,
             conv_b, wa, ba, wout_w, wout_b, wint_w, wint_b):
    bn = mol_repr.shape[0]
    bb = math.gcd(bn, 1024)
    nblk = bn // bb

    phar2 = phar_prompt.reshape(bn, NQ * DIM)
    prot_flat = protein_batch.reshape(nblk, bb, SEQ).transpose(0, 2, 1) \
                             .reshape(nblk, 1, SEQ * bb)

    band = _band_cat(conv_w)
    emb_ln = _layernorm(emb, prot_gamma, prot_beta)
    t1 = jnp.dot(emb_ln, band[0])
    packt = jnp.concatenate([
        t1.T,
        band[1].T, band[2].T,
        wa.T,
    ], axis=0).astype(jnp.bfloat16)

    matsn = jnp.concatenate([proj_w1, proj_w2],
                            axis=0).astype(jnp.bfloat16)
    vec = jnp.concatenate([
        proj_b1, proj_b2, mol_gamma, mol_beta,
        jnp.pad(wint_b, ((0, 0), (0, DIM - 1))),
    ], axis=0)
    vect = jnp.concatenate([
        ba.T,
        jnp.broadcast_to(conv_b[0], (DIM, 1)),
        jnp.broadcast_to(conv_b[1], (DIM, 1)),
        jnp.broadcast_to(conv_b[2], (DIM, 1)),
        jnp.zeros((DIM, 1), jnp.float32),
    ], axis=1)

    D2 = 2 * DIM
    headmats = jnp.concatenate(
        [wout_w[j].T for j in range(LOUT)], axis=0)
    bias_cols = jnp.concatenate(
        [wout_b[j].T for j in range(LOUT)] + [wint_w], axis=1)
    packh = jnp.concatenate([
        headmats,
        jnp.pad(bias_cols, ((0, 0), (0, D2 - (LOUT + 1)))),
    ], axis=0).astype(jnp.bfloat16)

    out = pl.pallas_call(
        _dti_block_kernel,
        out_shape=jax.ShapeDtypeStruct((1, bn), jnp.float32),
        grid=(nblk,),
        in_specs=[
            pl.BlockSpec((bb, NQ * DIM), lambda b: (b, 0)),
            pl.BlockSpec((bb, DIM), lambda b: (b, 0)),
            pl.BlockSpec((1, 1, SEQ * bb), lambda b: (b, 0, 0)),
            pl.BlockSpec((LCNN * KTAP * DIM + DIM, DIM), lambda b: (0, 0)),
            pl.BlockSpec(((NQ + 1) * DIM, DIM), lambda b: (0, 0)),
            pl.BlockSpec((5, DIM), lambda b: (0, 0)),
            pl.BlockSpec((DIM, 5), lambda b: (0, 0)),
            pl.BlockSpec((4 * D2, D2), lambda b: (0, 0)),
        ],
        out_specs=pl.BlockSpec((1, bb), lambda b: (0, b)),
        compiler_params=pltpu.CompilerParams(
            dimension_semantics=("parallel",)),
    )(phar2, mol_repr, prot_flat, packt, matsn, vec, vect, packh)
    return out.reshape(bn, 1)


def kernel(phar_prompt, mol_repr, protein_batch, proj_w1, proj_b1, proj_w2,
           proj_b2, emb, mol_gamma, mol_beta, prot_gamma, prot_beta, conv_w,
           conv_b, wa, ba, wout_w, wout_b, wint_w, wint_b):
    return _forward(phar_prompt, mol_repr, protein_batch, proj_w1, proj_b1,
                    proj_w2, proj_b2, emb, mol_gamma, mol_beta, prot_gamma,
                    prot_beta, conv_w, conv_b, wa, ba, wout_w, wout_b,
                    wint_w, wint_b)


# speedup vs baseline: 1.1926x; 1.0537x over previous
import math

import jax
import jax.numpy as jnp
import numpy as np
from jax import lax
from jax.experimental import pallas as pl
from jax.experimental.pallas import tpu as pltpu

SEQ = 16
DIM = 32
NQ = 3
NWORD = 32
WIN = 2
KTAP = 2 * WIN + 1
LCNN = 3
LOUT = 3
LN_EPS = 1e-5


def _layernorm(x, g, b):
    mu = jnp.mean(x, axis=-1, keepdims=True)
    var = jnp.mean((x - mu) ** 2, axis=-1, keepdims=True)
    return (x - mu) * lax.rsqrt(var + LN_EPS) * g + b


def _gelu(x):
    return 0.5 * x * (1.0 + lax.erf(x * 0.7071067811865476))


def _band_cat(conv_w):
    w = conv_w.reshape(LCNN, KTAP, KTAP)
    c = jnp.arange(DIM)[:, None]
    d = jnp.arange(DIM)[None, :]
    dj = c - d + WIN
    valid = (dj >= 0) & (dj < KTAP)
    djc = jnp.clip(dj, 0, KTAP - 1)
    band = jnp.where(valid[None, None], w[:, :, djc], 0.0)
    return band.transpose(0, 2, 1, 3).reshape(LCNN, DIM, KTAP * DIM)


def _shift_sum_t(g, bb):
    n = SEQ * bb
    acc = None
    for di in range(KTAP):
        sl = g[di * DIM:(di + 1) * DIM, :]
        sh = (di - WIN) * bb
        if sh > 0:
            z = jnp.zeros((DIM, sh), jnp.float32)
            t = jnp.concatenate([sl[:, sh:], z], axis=1)
        elif sh < 0:
            z = jnp.zeros((DIM, -sh), jnp.float32)
            t = jnp.concatenate([z, sl[:, :n + sh]], axis=1)
        else:
            t = sl
        acc = t if acc is None else acc + t
    return acc


def _dti_block_kernel(phar_ref, mol_ref, prot_ref, packt_ref, matsn_ref,
                      vec_ref, vect_ref, packh_ref, out_ref):
    bb = phar_ref.shape[0]
    n = SEQ * bb
    f32 = jnp.float32

    idx = prot_ref[0]
    iota = lax.broadcasted_iota(jnp.int32, (NWORD, n), 0)
    onehot = (idx == iota).astype(jnp.bfloat16)
    t1t = packt_ref[0:KTAP * DIM, :]
    g = jnp.dot(t1t, onehot, preferred_element_type=f32)
    ba_col = vect_ref[:, 0:1]
    xs = jnp.maximum(_shift_sum_t(g, bb) + vect_ref[:, 1:2], 0.0)
    xs = xs.astype(jnp.bfloat16)
    for l in range(1, LCNN):
        bt = packt_ref[l * KTAP * DIM:(l + 1) * KTAP * DIM, :]
        g = jnp.dot(bt, xs, preferred_element_type=f32)
        xs = jnp.maximum(_shift_sum_t(g, bb) + vect_ref[:, 1 + l:2 + l], 0.0)
        xs = xs.astype(jnp.bfloat16)

    p = phar_ref[...].astype(jnp.bfloat16)
    h1 = _gelu(jnp.dot(p, matsn_ref[0:NQ * DIM, :],
                       preferred_element_type=f32) + vec_ref[0:1, :])
    h1 = h1.astype(jnp.bfloat16)
    prompt = jnp.dot(h1, matsn_ref[NQ * DIM:NQ * DIM + DIM, :],
                     preferred_element_type=f32) + vec_ref[1:2, :]
    mol = _layernorm(prompt + mol_ref[...], vec_ref[2:3, :], vec_ref[3:4, :])
    molt = jnp.transpose(mol).astype(jnp.bfloat16)

    wat = packt_ref[LCNN * KTAP * DIM:LCNN * KTAP * DIM + DIM, :]
    ht = jnp.maximum(jnp.dot(wat, molt, preferred_element_type=f32)
                     + ba_col, 0.0)
    hst = jnp.maximum(jnp.dot(wat, xs, preferred_element_type=f32)
                      + ba_col, 0.0)
    ht_tiled = jnp.concatenate([ht] * SEQ, axis=1)
    ones_d = jnp.ones((1, DIM), f32)
    sig = jnp.dot(ones_d, ht_tiled * hst, preferred_element_type=f32)
    wts = jnp.tanh(sig)
    wprod = wts * hst
    prott = wprod[:, 0:bb]
    for s in range(1, SEQ):
        prott = prott + wprod[:, s * bb:(s + 1) * bb]
    prott = (prott * (1.0 / SEQ)).astype(jnp.bfloat16)

    D2 = 2 * DIM
    cat = jnp.maximum(
        jnp.dot(packh_ref[0:D2, 0:DIM], molt, preferred_element_type=f32)
        + jnp.dot(packh_ref[0:D2, DIM:D2], prott, preferred_element_type=f32)
        + packh_ref[LOUT * D2:LOUT * D2 + D2, 0:1].astype(f32), 0.0)
    cat = cat.astype(jnp.bfloat16)
    for j in range(1, LOUT):
        wjt = packh_ref[j * D2:(j + 1) * D2, :]
        cat = jnp.maximum(
            jnp.dot(wjt, cat, preferred_element_type=f32)
            + packh_ref[LOUT * D2:LOUT * D2 + D2, j:j + 1].astype(f32),
            0.0).astype(jnp.bfloat16)

    ones_2d = jnp.ones((1, D2), jnp.bfloat16)
    wint_col = packh_ref[LOUT * D2:LOUT * D2 + D2, LOUT:LOUT + 1]
    out = (jnp.dot(ones_2d, cat * wint_col, preferred_element_type=f32)
           + vec_ref[4:5, 0:1])
    out_ref[...] = out


@jax.jit
def _forward(phar_prompt, mol_repr, protein_batch, proj_w1, proj_b1, proj_w2,
             proj_b2, emb, mol_gamma, mol_beta, prot_gamma, prot_beta, conv_w,
             conv_b, wa, ba, wout_w, wout_b, wint_w, wint_b):
    bn = mol_repr.shape[0]
    bb = math.gcd(bn, 2048)
    nblk = bn // bb

    phar2 = phar_prompt.reshape(bn, NQ * DIM)
    prot_flat = protein_batch.reshape(nblk, bb, SEQ).transpose(0, 2, 1) \
                             .reshape(nblk, 1, SEQ * bb)

    band = _band_cat(conv_w)
    emb_ln = _layernorm(emb, prot_gamma, prot_beta)
    t1 = jnp.dot(emb_ln, band[0])
    packt = jnp.concatenate([
        t1.T,
        band[1].T, band[2].T,
        wa.T,
    ], axis=0).astype(jnp.bfloat16)

    matsn = jnp.concatenate([proj_w1, proj_w2],
                            axis=0).astype(jnp.bfloat16)
    vec = jnp.concatenate([
        proj_b1, proj_b2, mol_gamma, mol_beta,
        jnp.pad(wint_b, ((0, 0), (0, DIM - 1))),
    ], axis=0)
    vect = jnp.concatenate([
        ba.T,
        jnp.broadcast_to(conv_b[0], (DIM, 1)),
        jnp.broadcast_to(conv_b[1], (DIM, 1)),
        jnp.broadcast_to(conv_b[2], (DIM, 1)),
        jnp.zeros((DIM, 1), jnp.float32),
    ], axis=1)

    D2 = 2 * DIM
    headmats = jnp.concatenate(
        [wout_w[j].T for j in range(LOUT)], axis=0)
    bias_cols = jnp.concatenate(
        [wout_b[j].T for j in range(LOUT)] + [wint_w], axis=1)
    packh = jnp.concatenate([
        headmats,
        jnp.pad(bias_cols, ((0, 0), (0, D2 - (LOUT + 1)))),
    ], axis=0).astype(jnp.bfloat16)

    out = pl.pallas_call(
        _dti_block_kernel,
        out_shape=jax.ShapeDtypeStruct((1, bn), jnp.float32),
        grid=(nblk,),
        in_specs=[
            pl.BlockSpec((bb, NQ * DIM), lambda b: (b, 0)),
            pl.BlockSpec((bb, DIM), lambda b: (b, 0)),
            pl.BlockSpec((1, 1, SEQ * bb), lambda b: (b, 0, 0)),
            pl.BlockSpec((LCNN * KTAP * DIM + DIM, DIM), lambda b: (0, 0)),
            pl.BlockSpec(((NQ + 1) * DIM, DIM), lambda b: (0, 0)),
            pl.BlockSpec((5, DIM), lambda b: (0, 0)),
            pl.BlockSpec((DIM, 5), lambda b: (0, 0)),
            pl.BlockSpec((4 * D2, D2), lambda b: (0, 0)),
        ],
        out_specs=pl.BlockSpec((1, bb), lambda b: (0, b)),
        compiler_params=pltpu.CompilerParams(
            dimension_semantics=("parallel",)),
    )(phar2, mol_repr, prot_flat, packt, matsn, vec, vect, packh)
    return out.reshape(bn, 1)


def kernel(phar_prompt, mol_repr, protein_batch, proj_w1, proj_b1, proj_w2,
           proj_b2, emb, mol_gamma, mol_beta, prot_gamma, prot_beta, conv_w,
           conv_b, wa, ba, wout_w, wout_b, wint_w, wint_b):
    return _forward(phar_prompt, mol_repr, protein_batch, proj_w1, proj_b1,
                    proj_w2, proj_b2, emb, mol_gamma, mol_beta, prot_gamma,
                    prot_beta, conv_w, conv_b, wa, ba, wout_w, wout_b,
                    wint_w, wint_b)


# speedup vs baseline: 1.2379x; 1.0380x over previous
import math

import jax
import jax.numpy as jnp
import numpy as np
from jax import lax
from jax.experimental import pallas as pl
from jax.experimental.pallas import tpu as pltpu

SEQ = 16
DIM = 32
NQ = 3
NWORD = 32
WIN = 2
KTAP = 2 * WIN + 1
LCNN = 3
LOUT = 3
LN_EPS = 1e-5


def _layernorm(x, g, b):
    mu = jnp.mean(x, axis=-1, keepdims=True)
    var = jnp.mean((x - mu) ** 2, axis=-1, keepdims=True)
    return (x - mu) * lax.rsqrt(var + LN_EPS) * g + b


def _gelu(x):
    return 0.5 * x * (1.0 + lax.erf(x * 0.7071067811865476))


def _band_cat(conv_w):
    w = conv_w.reshape(LCNN, KTAP, KTAP)
    c = jnp.arange(DIM)[:, None]
    d = jnp.arange(DIM)[None, :]
    dj = c - d + WIN
    valid = (dj >= 0) & (dj < KTAP)
    djc = jnp.clip(dj, 0, KTAP - 1)
    band = jnp.where(valid[None, None], w[:, :, djc], 0.0)
    return band.transpose(0, 2, 1, 3).reshape(LCNN, DIM, KTAP * DIM)


def _shift_lanes(x, sh):
    if sh == 0:
        return x
    nl = x.shape[1]
    z = jnp.zeros((x.shape[0], abs(sh)), x.dtype)
    if sh > 0:
        return jnp.concatenate([x[:, sh:], z], axis=1)
    return jnp.concatenate([z, x[:, :nl + sh]], axis=1)


def _dti_block_kernel(phar_ref, mol_ref, prot_ref, packt_ref, matsn_ref,
                      vec_ref, vect_ref, packh_ref, out_ref):
    bb = phar_ref.shape[0]
    n = SEQ * bb
    f32 = jnp.float32

    idx = prot_ref[0]
    iota = lax.broadcasted_iota(jnp.int32, (NWORD, n), 0)
    xs = (idx == iota).astype(jnp.bfloat16)
    ba_col = vect_ref[:, 0:1]
    for l in range(LCNN):
        acc = None
        for di in range(KTAP):
            r0 = (l * KTAP + di) * DIM
            w = packt_ref[r0:r0 + DIM, :]
            t = _shift_lanes(jnp.dot(w, xs, preferred_element_type=f32),
                             (di - WIN) * bb)
            acc = t if acc is None else acc + t
        xs = jnp.maximum(acc + vect_ref[:, 1 + l:2 + l], 0.0)
        xs = xs.astype(jnp.bfloat16)

    p = phar_ref[...].astype(jnp.bfloat16)
    h1 = _gelu(jnp.dot(p, matsn_ref[0:NQ * DIM, :],
                       preferred_element_type=f32) + vec_ref[0:1, :])
    h1 = h1.astype(jnp.bfloat16)
    prompt = jnp.dot(h1, matsn_ref[NQ * DIM:NQ * DIM + DIM, :],
                     preferred_element_type=f32) + vec_ref[1:2, :]
    mol = _layernorm(prompt + mol_ref[...], vec_ref[2:3, :], vec_ref[3:4, :])
    molt = jnp.transpose(mol).astype(jnp.bfloat16)

    wat = packt_ref[LCNN * KTAP * DIM:LCNN * KTAP * DIM + DIM, :]
    ht = jnp.maximum(jnp.dot(wat, molt, preferred_element_type=f32)
                     + ba_col, 0.0)
    hst = jnp.maximum(jnp.dot(wat, xs, preferred_element_type=f32)
                      + ba_col, 0.0)
    ht_tiled = jnp.concatenate([ht] * SEQ, axis=1)
    ones_d = jnp.ones((1, DIM), f32)
    sig = jnp.dot(ones_d, ht_tiled * hst, preferred_element_type=f32)
    wts = jnp.tanh(sig)
    wprod = wts * hst
    prott = wprod[:, 0:bb]
    for s in range(1, SEQ):
        prott = prott + wprod[:, s * bb:(s + 1) * bb]
    prott = (prott * (1.0 / SEQ)).astype(jnp.bfloat16)

    D2 = 2 * DIM
    cat = jnp.maximum(
        jnp.dot(packh_ref[0:D2, 0:DIM], molt, preferred_element_type=f32)
        + jnp.dot(packh_ref[0:D2, DIM:D2], prott, preferred_element_type=f32)
        + packh_ref[LOUT * D2:LOUT * D2 + D2, 0:1].astype(f32), 0.0)
    cat = cat.astype(jnp.bfloat16)
    for j in range(1, LOUT):
        wjt = packh_ref[j * D2:(j + 1) * D2, :]
        cat = jnp.maximum(
            jnp.dot(wjt, cat, preferred_element_type=f32)
            + packh_ref[LOUT * D2:LOUT * D2 + D2, j:j + 1].astype(f32),
            0.0).astype(jnp.bfloat16)

    ones_2d = jnp.ones((1, D2), jnp.bfloat16)
    wint_col = packh_ref[LOUT * D2:LOUT * D2 + D2, LOUT:LOUT + 1]
    out = (jnp.dot(ones_2d, cat * wint_col, preferred_element_type=f32)
           + vec_ref[4:5, 0:1])
    out_ref[...] = out


@jax.jit
def _forward(phar_prompt, mol_repr, protein_batch, proj_w1, proj_b1, proj_w2,
             proj_b2, emb, mol_gamma, mol_beta, prot_gamma, prot_beta, conv_w,
             conv_b, wa, ba, wout_w, wout_b, wint_w, wint_b):
    bn = mol_repr.shape[0]
    bb = math.gcd(bn, 2048)
    nblk = bn // bb

    phar2 = phar_prompt.reshape(bn, NQ * DIM)
    prot_flat = protein_batch.reshape(nblk, bb, SEQ).transpose(0, 2, 1) \
                             .reshape(nblk, 1, SEQ * bb)

    band = _band_cat(conv_w)
    emb_ln = _layernorm(emb, prot_gamma, prot_beta)
    t1 = jnp.dot(emb_ln, band[0])
    taps = []
    for l in range(LCNN):
        base = t1 if l == 0 else band[l]
        for di in range(KTAP):
            taps.append(base[:, di * DIM:(di + 1) * DIM].T)
    packt = jnp.concatenate(taps + [wa.T],
                            axis=0).astype(jnp.bfloat16)

    matsn = jnp.concatenate([proj_w1, proj_w2],
                            axis=0).astype(jnp.bfloat16)
    vec = jnp.concatenate([
        proj_b1, proj_b2, mol_gamma, mol_beta,
        jnp.pad(wint_b, ((0, 0), (0, DIM - 1))),
    ], axis=0)
    vect = jnp.concatenate([
        ba.T,
        jnp.broadcast_to(conv_b[0], (DIM, 1)),
        jnp.broadcast_to(conv_b[1], (DIM, 1)),
        jnp.broadcast_to(conv_b[2], (DIM, 1)),
        jnp.zeros((DIM, 1), jnp.float32),
    ], axis=1)

    D2 = 2 * DIM
    headmats = jnp.concatenate(
        [wout_w[j].T for j in range(LOUT)], axis=0)
    bias_cols = jnp.concatenate(
        [wout_b[j].T for j in range(LOUT)] + [wint_w], axis=1)
    packh = jnp.concatenate([
        headmats,
        jnp.pad(bias_cols, ((0, 0), (0, D2 - (LOUT + 1)))),
    ], axis=0).astype(jnp.bfloat16)

    out = pl.pallas_call(
        _dti_block_kernel,
        out_shape=jax.ShapeDtypeStruct((1, bn), jnp.float32),
        grid=(nblk,),
        in_specs=[
            pl.BlockSpec((bb, NQ * DIM), lambda b: (b, 0)),
            pl.BlockSpec((bb, DIM), lambda b: (b, 0)),
            pl.BlockSpec((1, 1, SEQ * bb), lambda b: (b, 0, 0)),
            pl.BlockSpec((LCNN * KTAP * DIM + DIM, DIM), lambda b: (0, 0)),
            pl.BlockSpec(((NQ + 1) * DIM, DIM), lambda b: (0, 0)),
            pl.BlockSpec((5, DIM), lambda b: (0, 0)),
            pl.BlockSpec((DIM, 5), lambda b: (0, 0)),
            pl.BlockSpec((4 * D2, D2), lambda b: (0, 0)),
        ],
        out_specs=pl.BlockSpec((1, bb), lambda b: (0, b)),
        compiler_params=pltpu.CompilerParams(
            dimension_semantics=("parallel",)),
    )(phar2, mol_repr, prot_flat, packt, matsn, vec, vect, packh)
    return out.reshape(bn, 1)


def kernel(phar_prompt, mol_repr, protein_batch, proj_w1, proj_b1, proj_w2,
           proj_b2, emb, mol_gamma, mol_beta, prot_gamma, prot_beta, conv_w,
           conv_b, wa, ba, wout_w, wout_b, wint_w, wint_b):
    return _forward(phar_prompt, mol_repr, protein_batch, proj_w1, proj_b1,
                    proj_w2, proj_b2, emb, mol_gamma, mol_beta, prot_gamma,
                    prot_beta, conv_w, conv_b, wa, ba, wout_w, wout_b,
                    wint_w, wint_b)


# speedup vs baseline: 1.6747x; 1.3529x over previous
import math

import jax
import jax.numpy as jnp
import numpy as np
from jax import lax
from jax.experimental import pallas as pl
from jax.experimental.pallas import tpu as pltpu

SEQ = 16
DIM = 32
NQ = 3
NWORD = 32
WIN = 2
KTAP = 2 * WIN + 1
LCNN = 3
LOUT = 3
LN_EPS = 1e-5


def _layernorm(x, g, b):
    mu = jnp.mean(x, axis=-1, keepdims=True)
    var = jnp.mean((x - mu) ** 2, axis=-1, keepdims=True)
    return (x - mu) * lax.rsqrt(var + LN_EPS) * g + b


def _gelu(x):
    return 0.5 * x * (1.0 + lax.erf(x * 0.7071067811865476))


def _band_cat(conv_w):
    w = conv_w.reshape(LCNN, KTAP, KTAP)
    c = jnp.arange(DIM)[:, None]
    d = jnp.arange(DIM)[None, :]
    dj = c - d + WIN
    valid = (dj >= 0) & (dj < KTAP)
    djc = jnp.clip(dj, 0, KTAP - 1)
    band = jnp.where(valid[None, None], w[:, :, djc], 0.0)
    return band.transpose(0, 2, 1, 3).reshape(LCNN, DIM, KTAP * DIM)


def _shift_lanes(x, sh):
    if sh == 0:
        return x
    nl = x.shape[1]
    z = jnp.zeros((x.shape[0], abs(sh)), x.dtype)
    if sh > 0:
        return jnp.concatenate([x[:, sh:], z], axis=1)
    return jnp.concatenate([z, x[:, :nl + sh]], axis=1)


def _dti_block_kernel(phar_ref, mol_ref, prot_ref, packc_ref, wat_ref,
                      matsn_ref, vec_ref, vect_ref, packh_ref, out_ref):
    bb = phar_ref.shape[0]
    n = SEQ * bb
    f32 = jnp.float32

    idx = prot_ref[0]
    iota = lax.broadcasted_iota(jnp.int32, (NWORD, n), 0)
    xs = (idx == iota).astype(jnp.bfloat16)
    ba_col = vect_ref[:, 0:1]
    for l in range(LCNN):
        stack = jnp.concatenate(
            [_shift_lanes(xs, (di - WIN) * bb) for di in range(KTAP)],
            axis=0)
        wc = packc_ref[l * DIM:(l + 1) * DIM, :]
        xs = jnp.maximum(
            jnp.dot(wc, stack, preferred_element_type=f32)
            + vect_ref[:, 1 + l:2 + l], 0.0)
        xs = xs.astype(jnp.bfloat16)

    p = phar_ref[...].astype(jnp.bfloat16)
    h1 = _gelu(jnp.dot(p, matsn_ref[0:NQ * DIM, :],
                       preferred_element_type=f32) + vec_ref[0:1, :])
    h1 = h1.astype(jnp.bfloat16)
    prompt = jnp.dot(h1, matsn_ref[NQ * DIM:NQ * DIM + DIM, :],
                     preferred_element_type=f32) + vec_ref[1:2, :]
    mol = _layernorm(prompt + mol_ref[...], vec_ref[2:3, :], vec_ref[3:4, :])
    molt = jnp.transpose(mol).astype(jnp.bfloat16)

    wat = wat_ref[...]
    ht = jnp.maximum(jnp.dot(wat, molt, preferred_element_type=f32)
                     + ba_col, 0.0)
    hst = jnp.maximum(jnp.dot(wat, xs, preferred_element_type=f32)
                      + ba_col, 0.0)
    ht_tiled = jnp.concatenate([ht] * SEQ, axis=1)
    ones_d = jnp.ones((1, DIM), f32)
    sig = jnp.dot(ones_d, ht_tiled * hst, preferred_element_type=f32)
    wts = jnp.tanh(sig)
    wprod = wts * hst
    prott = wprod[:, 0:bb]
    for s in range(1, SEQ):
        prott = prott + wprod[:, s * bb:(s + 1) * bb]
    prott = (prott * (1.0 / SEQ)).astype(jnp.bfloat16)

    D2 = 2 * DIM
    cat = jnp.maximum(
        jnp.dot(packh_ref[0:D2, 0:DIM], molt, preferred_element_type=f32)
        + jnp.dot(packh_ref[0:D2, DIM:D2], prott, preferred_element_type=f32)
        + packh_ref[LOUT * D2:LOUT * D2 + D2, 0:1].astype(f32), 0.0)
    cat = cat.astype(jnp.bfloat16)
    for j in range(1, LOUT):
        wjt = packh_ref[j * D2:(j + 1) * D2, :]
        cat = jnp.maximum(
            jnp.dot(wjt, cat, preferred_element_type=f32)
            + packh_ref[LOUT * D2:LOUT * D2 + D2, j:j + 1].astype(f32),
            0.0).astype(jnp.bfloat16)

    ones_2d = jnp.ones((1, D2), jnp.bfloat16)
    wint_col = packh_ref[LOUT * D2:LOUT * D2 + D2, LOUT:LOUT + 1]
    out = (jnp.dot(ones_2d, cat * wint_col, preferred_element_type=f32)
           + vec_ref[4:5, 0:1])
    out_ref[...] = out


@jax.jit
def _forward(phar_prompt, mol_repr, protein_batch, proj_w1, proj_b1, proj_w2,
             proj_b2, emb, mol_gamma, mol_beta, prot_gamma, prot_beta, conv_w,
             conv_b, wa, ba, wout_w, wout_b, wint_w, wint_b):
    bn = mol_repr.shape[0]
    bb = math.gcd(bn, 2048)
    nblk = bn // bb

    phar2 = phar_prompt.reshape(bn, NQ * DIM)
    prot_flat = protein_batch.reshape(nblk, bb, SEQ).transpose(0, 2, 1) \
                             .reshape(nblk, 1, SEQ * bb)

    band = _band_cat(conv_w)
    emb_ln = _layernorm(emb, prot_gamma, prot_beta)
    t1 = jnp.dot(emb_ln, band[0])
    rows = []
    for l in range(LCNN):
        base = t1 if l == 0 else band[l]
        rows.append(jnp.concatenate(
            [base[:, di * DIM:(di + 1) * DIM].T for di in range(KTAP)],
            axis=1))
    packc = jnp.concatenate(rows, axis=0).astype(jnp.bfloat16)
    wat = wa.T.astype(jnp.bfloat16)

    matsn = jnp.concatenate([proj_w1, proj_w2],
                            axis=0).astype(jnp.bfloat16)
    vec = jnp.concatenate([
        proj_b1, proj_b2, mol_gamma, mol_beta,
        jnp.pad(wint_b, ((0, 0), (0, DIM - 1))),
    ], axis=0)
    vect = jnp.concatenate([
        ba.T,
        jnp.broadcast_to(conv_b[0], (DIM, 1)),
        jnp.broadcast_to(conv_b[1], (DIM, 1)),
        jnp.broadcast_to(conv_b[2], (DIM, 1)),
        jnp.zeros((DIM, 1), jnp.float32),
    ], axis=1)

    D2 = 2 * DIM
    headmats = jnp.concatenate(
        [wout_w[j].T for j in range(LOUT)], axis=0)
    bias_cols = jnp.concatenate(
        [wout_b[j].T for j in range(LOUT)] + [wint_w], axis=1)
    packh = jnp.concatenate([
        headmats,
        jnp.pad(bias_cols, ((0, 0), (0, D2 - (LOUT + 1)))),
    ], axis=0).astype(jnp.bfloat16)

    out = pl.pallas_call(
        _dti_block_kernel,
        out_shape=jax.ShapeDtypeStruct((1, bn), jnp.float32),
        grid=(nblk,),
        in_specs=[
            pl.BlockSpec((bb, NQ * DIM), lambda b: (b, 0)),
            pl.BlockSpec((bb, DIM), lambda b: (b, 0)),
            pl.BlockSpec((1, 1, SEQ * bb), lambda b: (b, 0, 0)),
            pl.BlockSpec((LCNN * DIM, KTAP * DIM), lambda b: (0, 0)),
            pl.BlockSpec((DIM, DIM), lambda b: (0, 0)),
            pl.BlockSpec(((NQ + 1) * DIM, DIM), lambda b: (0, 0)),
            pl.BlockSpec((5, DIM), lambda b: (0, 0)),
            pl.BlockSpec((DIM, 5), lambda b: (0, 0)),
            pl.BlockSpec((4 * D2, D2), lambda b: (0, 0)),
        ],
        out_specs=pl.BlockSpec((1, bb), lambda b: (0, b)),
        compiler_params=pltpu.CompilerParams(
            dimension_semantics=("parallel",)),
    )(phar2, mol_repr, prot_flat, packc, wat, matsn, vec, vect, packh)
    return out.reshape(bn, 1)


def kernel(phar_prompt, mol_repr, protein_batch, proj_w1, proj_b1, proj_w2,
           proj_b2, emb, mol_gamma, mol_beta, prot_gamma, prot_beta, conv_w,
           conv_b, wa, ba, wout_w, wout_b, wint_w, wint_b):
    return _forward(phar_prompt, mol_repr, protein_batch, proj_w1, proj_b1,
                    proj_w2, proj_b2, emb, mol_gamma, mol_beta, prot_gamma,
                    prot_beta, conv_w, conv_b, wa, ba, wout_w, wout_b,
                    wint_w, wint_b)


# speedup vs baseline: 1.7270x; 1.0312x over previous
import math

import jax
import jax.numpy as jnp
import numpy as np
from jax import lax
from jax.experimental import pallas as pl
from jax.experimental.pallas import tpu as pltpu

SEQ = 16
DIM = 32
NQ = 3
NWORD = 32
WIN = 2
KTAP = 2 * WIN + 1
LCNN = 3
LOUT = 3
LN_EPS = 1e-5


def _layernorm(x, g, b):
    mu = jnp.mean(x, axis=-1, keepdims=True)
    var = jnp.mean((x - mu) ** 2, axis=-1, keepdims=True)
    return (x - mu) * lax.rsqrt(var + LN_EPS) * g + b


def _gelu(x):
    return 0.5 * x * (1.0 + lax.erf(x * 0.7071067811865476))


def _band_cat(conv_w):
    w = conv_w.reshape(LCNN, KTAP, KTAP)
    c = jnp.arange(DIM)[:, None]
    d = jnp.arange(DIM)[None, :]
    dj = c - d + WIN
    valid = (dj >= 0) & (dj < KTAP)
    djc = jnp.clip(dj, 0, KTAP - 1)
    band = jnp.where(valid[None, None], w[:, :, djc], 0.0)
    return band.transpose(0, 2, 1, 3).reshape(LCNN, DIM, KTAP * DIM)


def _shift_lanes(x, sh):
    if sh == 0:
        return x
    nl = x.shape[1]
    z = jnp.zeros((x.shape[0], abs(sh)), x.dtype)
    if sh > 0:
        return jnp.concatenate([x[:, sh:], z], axis=1)
    return jnp.concatenate([z, x[:, :nl + sh]], axis=1)


def _dti_block_kernel(phar_ref, mol_ref, prot_ref, packc_ref, wat_ref,
                      matsn_ref, vec_ref, vect_ref, packh_ref, out_ref):
    bb = phar_ref.shape[0]
    n = SEQ * bb
    f32 = jnp.float32

    idx = prot_ref[0]
    iota = lax.broadcasted_iota(jnp.int32, (NWORD, n), 0)
    xs = (idx == iota).astype(f32)
    ba_col = vect_ref[:, 0:1]
    for l in range(LCNN):
        stack = jnp.concatenate(
            [_shift_lanes(xs, (di - WIN) * bb) for di in range(KTAP)],
            axis=0)
        wc = packc_ref[l * DIM:(l + 1) * DIM, :]
        xs = jnp.maximum(
            jnp.dot(wc, stack, preferred_element_type=f32)
            + vect_ref[:, 1 + l:2 + l], 0.0)

    p = phar_ref[...]
    h1 = _gelu(jnp.dot(p, matsn_ref[0:NQ * DIM, :],
                       preferred_element_type=f32) + vec_ref[0:1, :])
    prompt = jnp.dot(h1, matsn_ref[NQ * DIM:NQ * DIM + DIM, :],
                     preferred_element_type=f32) + vec_ref[1:2, :]
    mol = _layernorm(prompt + mol_ref[...], vec_ref[2:3, :], vec_ref[3:4, :])
    molt = jnp.transpose(mol)

    wat = wat_ref[...]
    ht = jnp.maximum(jnp.dot(wat, molt, preferred_element_type=f32)
                     + ba_col, 0.0)
    hst = jnp.maximum(jnp.dot(wat, xs, preferred_element_type=f32)
                      + ba_col, 0.0)
    ht_tiled = jnp.concatenate([ht] * SEQ, axis=1)
    ones_d = jnp.ones((1, DIM), f32)
    sig = jnp.dot(ones_d, ht_tiled * hst, preferred_element_type=f32)
    wts = jnp.tanh(sig)
    wprod = wts * hst
    prott = wprod[:, 0:bb]
    for s in range(1, SEQ):
        prott = prott + wprod[:, s * bb:(s + 1) * bb]
    prott = prott * (1.0 / SEQ)

    D2 = 2 * DIM
    cat = jnp.maximum(
        jnp.dot(packh_ref[0:D2, 0:DIM], molt, preferred_element_type=f32)
        + jnp.dot(packh_ref[0:D2, DIM:D2], prott, preferred_element_type=f32)
        + packh_ref[LOUT * D2:LOUT * D2 + D2, 0:1], 0.0)
    for j in range(1, LOUT):
        wjt = packh_ref[j * D2:(j + 1) * D2, :]
        cat = jnp.maximum(
            jnp.dot(wjt, cat, preferred_element_type=f32)
            + packh_ref[LOUT * D2:LOUT * D2 + D2, j:j + 1],
            0.0)

    ones_2d = jnp.ones((1, D2), f32)
    wint_col = packh_ref[LOUT * D2:LOUT * D2 + D2, LOUT:LOUT + 1]
    out = (jnp.dot(ones_2d, cat * wint_col, preferred_element_type=f32)
           + vec_ref[4:5, 0:1])
    out_ref[...] = out


@jax.jit
def _forward(phar_prompt, mol_repr, protein_batch, proj_w1, proj_b1, proj_w2,
             proj_b2, emb, mol_gamma, mol_beta, prot_gamma, prot_beta, conv_w,
             conv_b, wa, ba, wout_w, wout_b, wint_w, wint_b):
    bn = mol_repr.shape[0]
    bb = math.gcd(bn, 2048)
    nblk = bn // bb

    phar2 = phar_prompt.reshape(bn, NQ * DIM)
    prot_flat = protein_batch.reshape(nblk, bb, SEQ).transpose(0, 2, 1) \
                             .reshape(nblk, 1, SEQ * bb)

    band = _band_cat(conv_w)
    emb_ln = _layernorm(emb, prot_gamma, prot_beta)
    t1 = jnp.dot(emb_ln, band[0])
    rows = []
    for l in range(LCNN):
        base = t1 if l == 0 else band[l]
        rows.append(jnp.concatenate(
            [base[:, di * DIM:(di + 1) * DIM].T for di in range(KTAP)],
            axis=1))
    packc = jnp.concatenate(rows, axis=0)
    wat = wa.T

    matsn = jnp.concatenate([proj_w1, proj_w2],
                            axis=0)
    vec = jnp.concatenate([
        proj_b1, proj_b2, mol_gamma, mol_beta,
        jnp.pad(wint_b, ((0, 0), (0, DIM - 1))),
    ], axis=0)
    vect = jnp.concatenate([
        ba.T,
        jnp.broadcast_to(conv_b[0], (DIM, 1)),
        jnp.broadcast_to(conv_b[1], (DIM, 1)),
        jnp.broadcast_to(conv_b[2], (DIM, 1)),
        jnp.zeros((DIM, 1), jnp.float32),
    ], axis=1)

    D2 = 2 * DIM
    headmats = jnp.concatenate(
        [wout_w[j].T for j in range(LOUT)], axis=0)
    bias_cols = jnp.concatenate(
        [wout_b[j].T for j in range(LOUT)] + [wint_w], axis=1)
    packh = jnp.concatenate([
        headmats,
        jnp.pad(bias_cols, ((0, 0), (0, D2 - (LOUT + 1)))),
    ], axis=0)

    out = pl.pallas_call(
        _dti_block_kernel,
        out_shape=jax.ShapeDtypeStruct((1, bn), jnp.float32),
        grid=(nblk,),
        in_specs=[
            pl.BlockSpec((bb, NQ * DIM), lambda b: (b, 0)),
            pl.BlockSpec((bb, DIM), lambda b: (b, 0)),
            pl.BlockSpec((1, 1, SEQ * bb), lambda b: (b, 0, 0)),
            pl.BlockSpec((LCNN * DIM, KTAP * DIM), lambda b: (0, 0)),
            pl.BlockSpec((DIM, DIM), lambda b: (0, 0)),
            pl.BlockSpec(((NQ + 1) * DIM, DIM), lambda b: (0, 0)),
            pl.BlockSpec((5, DIM), lambda b: (0, 0)),
            pl.BlockSpec((DIM, 5), lambda b: (0, 0)),
            pl.BlockSpec((4 * D2, D2), lambda b: (0, 0)),
        ],
        out_specs=pl.BlockSpec((1, bb), lambda b: (0, b)),
        compiler_params=pltpu.CompilerParams(
            dimension_semantics=("parallel",)),
    )(phar2, mol_repr, prot_flat, packc, wat, matsn, vec, vect, packh)
    return out.reshape(bn, 1)


def kernel(phar_prompt, mol_repr, protein_batch, proj_w1, proj_b1, proj_w2,
           proj_b2, emb, mol_gamma, mol_beta, prot_gamma, prot_beta, conv_w,
           conv_b, wa, ba, wout_w, wout_b, wint_w, wint_b):
    return _forward(phar_prompt, mol_repr, protein_batch, proj_w1, proj_b1,
                    proj_w2, proj_b2, emb, mol_gamma, mol_beta, prot_gamma,
                    prot_beta, conv_w, conv_b, wa, ba, wout_w, wout_b,
                    wint_w, wint_b)
